# baseline (speedup 1.0000x reference)
import functools

import jax
import jax.numpy as jnp
from jax import lax
from jax.experimental import pallas as pl
from jax.experimental.pallas import tpu as pltpu
from jax.experimental.pallas import tpu_sc as plsc

NUM_PROPS = 8192
NUM_RELS = 65536
EMB_DIM = 200
HID = 1024

NC = 2
NS = 16
NW = NC * NS

ROWS_PER_W = NUM_RELS // NW


_SG_CHUNK = 128
_PROP_PAD = 128


def _sc_gather_props(prop128, sub_idx, obj_idx):
    mesh = plsc.VectorSubcoreMesh(core_axis_name="c", subcore_axis_name="s")

    @functools.partial(
        pl.kernel,
        out_type=[
            jax.ShapeDtypeStruct((NUM_RELS // 8, _PROP_PAD), jnp.float32),
            jax.ShapeDtypeStruct((NUM_RELS // 8, _PROP_PAD), jnp.float32),
        ],
        mesh=mesh,
        scratch_types=[
            pltpu.VMEM((_SG_CHUNK,), jnp.int32),
            pltpu.VMEM((_SG_CHUNK,), jnp.int32),
            pltpu.VMEM((_SG_CHUNK, _PROP_PAD), jnp.float32),
            pltpu.VMEM((_SG_CHUNK, _PROP_PAD), jnp.float32),
            pltpu.VMEM((_SG_CHUNK // 8, _PROP_PAD), jnp.float32),
            pltpu.VMEM((_SG_CHUNK // 8, _PROP_PAD), jnp.float32),
            pltpu.SemaphoreType.DMA,
            pltpu.SemaphoreType.DMA,
        ],
    )
    def k(prop_hbm, sub_hbm, obj_hbm, osub_hbm, oobj_hbm,
          idxs_v, idxo_v, bufs_v, bufo_v, cbufs_v, cbufo_v, sem_s, sem_o):
        wid = lax.axis_index("s") * NC + lax.axis_index("c")
        base0 = wid * ROWS_PER_W

        def body(j, _):
            base = base0 + j * _SG_CHUNK
            pltpu.sync_copy(sub_hbm.at[pl.ds(base, _SG_CHUNK)], idxs_v)
            pltpu.sync_copy(obj_hbm.at[pl.ds(base, _SG_CHUNK)], idxo_v)
            cs = pltpu.async_copy(prop_hbm.at[idxs_v], bufs_v, sem_s)
            co = pltpu.async_copy(prop_hbm.at[idxo_v], bufo_v, sem_o)
            cs.wait()
            co.wait()

            def crow(g, carry):
                for kk in range(8):
                    cbufs_v[g, kk * 16:(kk + 1) * 16] = bufs_v[8 * g + kk, 0:16]
                    cbufo_v[g, kk * 16:(kk + 1) * 16] = bufo_v[8 * g + kk, 0:16]
                return carry

            lax.fori_loop(0, _SG_CHUNK // 8, crow, 0)
            base8 = pl.multiple_of(
                wid * (ROWS_PER_W // 8) + j * (_SG_CHUNK // 8), _SG_CHUNK // 8)
            pltpu.sync_copy(cbufs_v,
                            osub_hbm.at[pl.ds(base8, _SG_CHUNK // 8)])
            pltpu.sync_copy(cbufo_v,
                            oobj_hbm.at[pl.ds(base8, _SG_CHUNK // 8)])
            return _

        lax.fori_loop(0, ROWS_PER_W // _SG_CHUNK, body, 0)

    return k(prop128, sub_idx, obj_idx)


def _tc_tables(corpus, W_sub, b_sub, W_obj, b_obj):
    blk = 1024
    grid = NUM_PROPS // blk

    def body(x_ref, ws_ref, bs_ref, wo_ref, bo_ref, ts_ref, to_ref):
        x = x_ref[...]
        ts_ref[...] = jnp.maximum(
            jnp.dot(x, ws_ref[...], preferred_element_type=jnp.float32)
            + bs_ref[...],
            0.0,
        )
        to_ref[...] = jnp.maximum(
            jnp.dot(x, wo_ref[...], preferred_element_type=jnp.float32)
            + bo_ref[...],
            0.0,
        )

    return pl.pallas_call(
        body,
        grid=(grid,),
        in_specs=[
            pl.BlockSpec((blk, EMB_DIM), lambda i: (i, 0)),
            pl.BlockSpec((EMB_DIM, HID), lambda i: (0, 0)),
            pl.BlockSpec((1, HID), lambda i: (0, 0)),
            pl.BlockSpec((EMB_DIM, HID), lambda i: (0, 0)),
            pl.BlockSpec((1, HID), lambda i: (0, 0)),
        ],
        out_specs=[
            pl.BlockSpec((blk, HID), lambda i: (i, 0)),
            pl.BlockSpec((blk, HID), lambda i: (i, 0)),
        ],
        out_shape=[
            jax.ShapeDtypeStruct((NUM_PROPS, HID), jnp.float32),
            jax.ShapeDtypeStruct((NUM_PROPS, HID), jnp.float32),
        ],
    )(corpus, W_sub, b_sub.reshape(1, HID), W_obj, b_obj.reshape(1, HID))


def _tc_pos_into(partial_out, ps, po, rel8, W1p, W2p, W3p, b_int):
    blk = 2048
    grid = NUM_RELS // blk

    def body(buf_ref, x1_ref, x2_ref, x3_ref, w1_ref, w2_ref, w3_ref, b_ref,
             o_ref, acc_ref, sem):
        i = pl.program_id(0)
        slot = lax.rem(i, 2)

        @pl.when(i >= 2)
        def _():
            pltpu.make_async_copy(
                acc_ref.at[slot],
                o_ref.at[pl.ds((i - 2) * blk, blk), pl.ds(2 * HID, HID)],
                sem,
            ).wait()

        acc = jnp.dot(x1_ref[...], w1_ref[...], preferred_element_type=jnp.float32)
        acc += jnp.dot(x2_ref[...], w2_ref[...], preferred_element_type=jnp.float32)
        acc += jnp.dot(x3_ref[...], w3_ref[...], preferred_element_type=jnp.float32)
        acc_ref[slot] = jnp.maximum(acc + b_ref[...], 0.0)

        pltpu.make_async_copy(
            acc_ref.at[slot],
            o_ref.at[pl.ds(i * blk, blk), pl.ds(2 * HID, HID)],
            sem,
        ).start()

        @pl.when(i == grid - 1)
        def _():
            for back in (1, 0):
                pltpu.make_async_copy(
                    acc_ref.at[slot],
                    o_ref.at[pl.ds((i - back) * blk, blk), pl.ds(2 * HID, HID)],
                    sem,
                ).wait()

    return pl.pallas_call(
        body,
        grid=(grid,),
        in_specs=[
            pl.BlockSpec(memory_space=pl.ANY),
            pl.BlockSpec((blk, 16), lambda i: (i, 0)),
            pl.BlockSpec((blk, 16), lambda i: (i, 0)),
            pl.BlockSpec((blk, 8), lambda i: (i, 0)),
            pl.BlockSpec((16, HID), lambda i: (0, 0)),
            pl.BlockSpec((16, HID), lambda i: (0, 0)),
            pl.BlockSpec((8, HID), lambda i: (0, 0)),
            pl.BlockSpec((1, HID), lambda i: (0, 0)),
        ],
        out_specs=pl.BlockSpec(memory_space=pl.ANY),
        out_shape=jax.ShapeDtypeStruct((NUM_RELS, 3 * HID), jnp.float32),
        scratch_shapes=[
            pltpu.VMEM((2, blk, HID), jnp.float32),
            pltpu.SemaphoreType.DMA,
        ],
        input_output_aliases={0: 0},
    )(partial_out, ps, po, rel8, W1p, W2p, W3p, b_int.reshape(1, HID))


_AS_CHUNK = 16
_AS_UNROLL = 16


def _sc_assemble(tsub, tobj, sub_idx, obj_idx):
    mesh = plsc.VectorSubcoreMesh(core_axis_name="c", subcore_axis_name="s")
    n_chunks = ROWS_PER_W // _AS_CHUNK

    @functools.partial(
        pl.kernel,
        out_type=jax.ShapeDtypeStruct((NUM_RELS, 3 * HID), jnp.float32),
        mesh=mesh,
        scratch_types=[
            pltpu.VMEM((ROWS_PER_W,), jnp.int32),
            pltpu.VMEM((ROWS_PER_W,), jnp.int32),
            pltpu.VMEM((2, _AS_CHUNK, HID), jnp.float32),
            pltpu.VMEM((2, _AS_CHUNK, HID), jnp.float32),
            pltpu.SemaphoreType.DMA,
            pltpu.SemaphoreType.DMA,
            pltpu.SemaphoreType.DMA,
            pltpu.SemaphoreType.DMA,
            pltpu.SemaphoreType.DMA,
            pltpu.SemaphoreType.DMA,
            pltpu.SemaphoreType.DMA,
            pltpu.SemaphoreType.DMA,
        ],
    )
    def k(tsub_hbm, tobj_hbm, sub_hbm, obj_hbm, out_hbm,
          idxs_v, idxo_v, bs_v, bo_v,
          gs0, gs1, go0, go1, ws0, ws1, wo0, wo1):
        wid = lax.axis_index("s") * NC + lax.axis_index("c")
        base0 = wid * ROWS_PER_W
        pltpu.sync_copy(sub_hbm.at[pl.ds(base0, ROWS_PER_W)], idxs_v)
        pltpu.sync_copy(obj_hbm.at[pl.ds(base0, ROWS_PER_W)], idxo_v)
        gsem = (gs0, gs1)
        osem = (go0, go1)
        wsem = ((ws0, wo0), (ws1, wo1))

        def block(g, _):
            j0 = g * _AS_UNROLL
            gathers = [None] * _AS_UNROLL
            writes = [None] * _AS_UNROLL
            for u in range(_AS_UNROLL):
                s = u % 2
                off = (j0 + u) * _AS_CHUNK
                base = base0 + off
                if u >= 2:
                    for w in writes[u - 2]:
                        w.wait()
                gathers[u] = (
                    pltpu.async_copy(
                        tsub_hbm.at[idxs_v.at[pl.ds(off, _AS_CHUNK)]],
                        bs_v.at[s], gsem[s]),
                    pltpu.async_copy(
                        tobj_hbm.at[idxo_v.at[pl.ds(off, _AS_CHUNK)]],
                        bo_v.at[s], osem[s]),
                )
                if u >= 1:
                    sp = (u - 1) % 2
                    offp = (j0 + u - 1) * _AS_CHUNK
                    basep = base0 + offp
                    for gcp in gathers[u - 1]:
                        gcp.wait()
                    writes[u - 1] = (
                        pltpu.async_copy(
                            bs_v.at[sp],
                            out_hbm.at[pl.ds(basep, _AS_CHUNK), pl.ds(0, HID)],
                            wsem[sp][0]),
                        pltpu.async_copy(
                            bo_v.at[sp],
                            out_hbm.at[pl.ds(basep, _AS_CHUNK), pl.ds(HID, HID)],
                            wsem[sp][1]),
                    )
            u = _AS_UNROLL - 1
            s = u % 2
            off = (j0 + u) * _AS_CHUNK
            base = base0 + off
            for gcp in gathers[u]:
                gcp.wait()
            writes[u] = (
                pltpu.async_copy(
                    bs_v.at[s],
                    out_hbm.at[pl.ds(base, _AS_CHUNK), pl.ds(0, HID)],
                    wsem[s][0]),
                pltpu.async_copy(
                    bo_v.at[s],
                    out_hbm.at[pl.ds(base, _AS_CHUNK), pl.ds(HID, HID)],
                    wsem[s][1]),
            )
            for w in writes[u - 1]:
                w.wait()
            for w in writes[u]:
                w.wait()
            return _

        lax.fori_loop(0, n_chunks // _AS_UNROLL, block, 0)

    return k(tsub, tobj, sub_idx, obj_idx)


def kernel(wordembedding_corpus, rel_pair_idxs, prop_info, rel_info,
           W_sub, b_sub, W_obj, b_obj, W_int, b_int):
    idx = rel_pair_idxs.astype(jnp.int32)
    sub_idx = idx[:, 0]
    obj_idx = idx[:, 1]

    prop128 = jnp.pad(prop_info, ((0, 0), (0, _PROP_PAD - prop_info.shape[1])))
    ps8, po8 = _sc_gather_props(prop128, sub_idx, obj_idx)
    ps = ps8.reshape(NUM_RELS, 16)
    po = po8.reshape(NUM_RELS, 16)

    tsub, tobj = _tc_tables(wordembedding_corpus, W_sub, b_sub, W_obj, b_obj)

    W1p = jnp.zeros((16, HID), jnp.float32).at[:9].set(W_int[:9])
    W2p = jnp.zeros((16, HID), jnp.float32).at[:9].set(W_int[9:18])
    W3p = jnp.zeros((8, HID), jnp.float32).at[:2].set(W_int[18:20])
    rel8 = jnp.pad(rel_info, ((0, 0), (0, 6)))

    partial_out = _sc_assemble(tsub, tobj, sub_idx, obj_idx)
    return _tc_pos_into(partial_out, ps, po, rel8, W1p, W2p, W3p, b_int)


# speedup vs baseline: 1.0144x; 1.0144x over previous
import functools

import jax
import jax.numpy as jnp
from jax import lax
from jax.experimental import pallas as pl
from jax.experimental.pallas import tpu as pltpu
from jax.experimental.pallas import tpu_sc as plsc

NUM_PROPS = 8192
NUM_RELS = 65536
EMB_DIM = 200
HID = 1024

NC = 2
NS = 16
NW = NC * NS

ROWS_PER_W = NUM_RELS // NW


_SG_CHUNK = 128
_PROP_PAD = 128


def _sc_gather_props(prop128, sub_idx, obj_idx):
    mesh = plsc.VectorSubcoreMesh(core_axis_name="c", subcore_axis_name="s")

    @functools.partial(
        pl.kernel,
        out_type=[
            jax.ShapeDtypeStruct((NUM_RELS, _PROP_PAD), jnp.float32),
            jax.ShapeDtypeStruct((NUM_RELS, _PROP_PAD), jnp.float32),
        ],
        mesh=mesh,
        scratch_types=[
            pltpu.VMEM((_SG_CHUNK,), jnp.int32),
            pltpu.VMEM((_SG_CHUNK,), jnp.int32),
            pltpu.VMEM((_SG_CHUNK, _PROP_PAD), jnp.float32),
            pltpu.VMEM((_SG_CHUNK, _PROP_PAD), jnp.float32),
            pltpu.SemaphoreType.DMA,
            pltpu.SemaphoreType.DMA,
        ],
    )
    def k(prop_hbm, sub_hbm, obj_hbm, osub_hbm, oobj_hbm,
          idxs_v, idxo_v, bufs_v, bufo_v, sem_s, sem_o):
        wid = lax.axis_index("s") * NC + lax.axis_index("c")
        base0 = wid * ROWS_PER_W

        def body(j, _):
            base = base0 + j * _SG_CHUNK
            pltpu.sync_copy(sub_hbm.at[pl.ds(base, _SG_CHUNK)], idxs_v)
            pltpu.sync_copy(obj_hbm.at[pl.ds(base, _SG_CHUNK)], idxo_v)
            cs = pltpu.async_copy(prop_hbm.at[idxs_v], bufs_v, sem_s)
            co = pltpu.async_copy(prop_hbm.at[idxo_v], bufo_v, sem_o)
            cs.wait()
            pltpu.sync_copy(bufs_v, osub_hbm.at[pl.ds(base, _SG_CHUNK)])
            co.wait()
            pltpu.sync_copy(bufo_v, oobj_hbm.at[pl.ds(base, _SG_CHUNK)])
            return _

        lax.fori_loop(0, ROWS_PER_W // _SG_CHUNK, body, 0)

    return k(prop128, sub_idx, obj_idx)


def _tc_tables(corpus, W_sub, b_sub, W_obj, b_obj):
    blk = 1024
    grid = NUM_PROPS // blk

    def body(x_ref, ws_ref, bs_ref, wo_ref, bo_ref, ts_ref, to_ref):
        x = x_ref[...]
        ts_ref[...] = jnp.maximum(
            jnp.dot(x, ws_ref[...], preferred_element_type=jnp.float32)
            + bs_ref[...],
            0.0,
        )
        to_ref[...] = jnp.maximum(
            jnp.dot(x, wo_ref[...], preferred_element_type=jnp.float32)
            + bo_ref[...],
            0.0,
        )

    return pl.pallas_call(
        body,
        grid=(grid,),
        in_specs=[
            pl.BlockSpec((blk, EMB_DIM), lambda i: (i, 0)),
            pl.BlockSpec((EMB_DIM, HID), lambda i: (0, 0)),
            pl.BlockSpec((1, HID), lambda i: (0, 0)),
            pl.BlockSpec((EMB_DIM, HID), lambda i: (0, 0)),
            pl.BlockSpec((1, HID), lambda i: (0, 0)),
        ],
        out_specs=[
            pl.BlockSpec((blk, HID), lambda i: (i, 0)),
            pl.BlockSpec((blk, HID), lambda i: (i, 0)),
        ],
        out_shape=[
            jax.ShapeDtypeStruct((NUM_PROPS, HID), jnp.float32),
            jax.ShapeDtypeStruct((NUM_PROPS, HID), jnp.float32),
        ],
    )(corpus, W_sub, b_sub.reshape(1, HID), W_obj, b_obj.reshape(1, HID))


def _tc_pos_into(partial_out, ps, po, rel8, W1p, W2p, W3p, b_int):
    blk = 2048
    grid = NUM_RELS // blk

    def body(buf_ref, x1_ref, x2_ref, x3_ref, w1_ref, w2_ref, w3_ref, b_ref,
             o_ref, acc_ref, sem):
        i = pl.program_id(0)
        slot = lax.rem(i, 2)

        @pl.when(i >= 2)
        def _():
            pltpu.make_async_copy(
                acc_ref.at[slot],
                o_ref.at[pl.ds((i - 2) * blk, blk), pl.ds(2 * HID, HID)],
                sem,
            ).wait()

        acc = jnp.dot(x1_ref[...], w1_ref[...], preferred_element_type=jnp.float32)
        acc += jnp.dot(x2_ref[...], w2_ref[...], preferred_element_type=jnp.float32)
        acc += jnp.dot(x3_ref[...], w3_ref[...], preferred_element_type=jnp.float32)
        acc_ref[slot] = jnp.maximum(acc + b_ref[...], 0.0)

        pltpu.make_async_copy(
            acc_ref.at[slot],
            o_ref.at[pl.ds(i * blk, blk), pl.ds(2 * HID, HID)],
            sem,
        ).start()

        @pl.when(i == grid - 1)
        def _():
            for back in (1, 0):
                pltpu.make_async_copy(
                    acc_ref.at[slot],
                    o_ref.at[pl.ds((i - back) * blk, blk), pl.ds(2 * HID, HID)],
                    sem,
                ).wait()

    return pl.pallas_call(
        body,
        grid=(grid,),
        in_specs=[
            pl.BlockSpec(memory_space=pl.ANY),
            pl.BlockSpec((blk, _PROP_PAD), lambda i: (i, 0)),
            pl.BlockSpec((blk, _PROP_PAD), lambda i: (i, 0)),
            pl.BlockSpec((blk, 8), lambda i: (i, 0)),
            pl.BlockSpec((_PROP_PAD, HID), lambda i: (0, 0)),
            pl.BlockSpec((_PROP_PAD, HID), lambda i: (0, 0)),
            pl.BlockSpec((8, HID), lambda i: (0, 0)),
            pl.BlockSpec((1, HID), lambda i: (0, 0)),
        ],
        out_specs=pl.BlockSpec(memory_space=pl.ANY),
        out_shape=jax.ShapeDtypeStruct((NUM_RELS, 3 * HID), jnp.float32),
        scratch_shapes=[
            pltpu.VMEM((2, blk, HID), jnp.float32),
            pltpu.SemaphoreType.DMA,
        ],
        input_output_aliases={0: 0},
    )(partial_out, ps, po, rel8, W1p, W2p, W3p, b_int.reshape(1, HID))


_AS_CHUNK = 16
_AS_UNROLL = 16


def _sc_assemble(tsub, tobj, sub_idx, obj_idx):
    mesh = plsc.VectorSubcoreMesh(core_axis_name="c", subcore_axis_name="s")
    n_chunks = ROWS_PER_W // _AS_CHUNK
    NSLOT = 3

    @functools.partial(
        pl.kernel,
        out_type=jax.ShapeDtypeStruct((NUM_RELS, 3 * HID), jnp.float32),
        mesh=mesh,
        scratch_types=[
            pltpu.VMEM((ROWS_PER_W,), jnp.int32),
            pltpu.VMEM((ROWS_PER_W,), jnp.int32),
            pltpu.VMEM((NSLOT, _AS_CHUNK, HID), jnp.float32),
            pltpu.VMEM((NSLOT, _AS_CHUNK, HID), jnp.float32),
        ] + [pltpu.SemaphoreType.DMA] * (4 * NSLOT),
    )
    def k(tsub_hbm, tobj_hbm, sub_hbm, obj_hbm, out_hbm,
          idxs_v, idxo_v, bs_v, bo_v, *sems):
        wid = lax.axis_index("s") * NC + lax.axis_index("c")
        base0 = wid * ROWS_PER_W
        pltpu.sync_copy(sub_hbm.at[pl.ds(base0, ROWS_PER_W)], idxs_v)
        pltpu.sync_copy(obj_hbm.at[pl.ds(base0, ROWS_PER_W)], idxo_v)
        gsem = sems[0:NSLOT]
        osem = sems[NSLOT:2 * NSLOT]
        wssem = sems[2 * NSLOT:3 * NSLOT]
        wosem = sems[3 * NSLOT:4 * NSLOT]

        def fire_gather(j, s):
            off = j * _AS_CHUNK
            return (
                pltpu.async_copy(
                    tsub_hbm.at[idxs_v.at[pl.ds(off, _AS_CHUNK)]],
                    bs_v.at[s], gsem[s]),
                pltpu.async_copy(
                    tobj_hbm.at[idxo_v.at[pl.ds(off, _AS_CHUNK)]],
                    bo_v.at[s], osem[s]),
            )

        def fire_write(j, s):
            base = base0 + j * _AS_CHUNK
            return (
                pltpu.async_copy(
                    bs_v.at[s],
                    out_hbm.at[pl.ds(base, _AS_CHUNK), pl.ds(0, HID)],
                    wssem[s]),
                pltpu.async_copy(
                    bo_v.at[s],
                    out_hbm.at[pl.ds(base, _AS_CHUNK), pl.ds(HID, HID)],
                    wosem[s]),
            )

        def block(g, _):
            j0 = g * _AS_UNROLL
            gathers = [None] * _AS_UNROLL
            writes = [None] * _AS_UNROLL
            for u in range(_AS_UNROLL):
                s = u % NSLOT
                if u >= NSLOT:
                    for w in writes[u - NSLOT]:
                        w.wait()
                gathers[u] = fire_gather(j0 + u, s)
                if u >= 1:
                    for gcp in gathers[u - 1]:
                        gcp.wait()
                    writes[u - 1] = fire_write(j0 + u - 1, (u - 1) % NSLOT)
            u = _AS_UNROLL - 1
            for gcp in gathers[u]:
                gcp.wait()
            writes[u] = fire_write(j0 + u, u % NSLOT)
            for uu in range(_AS_UNROLL - NSLOT + 1, _AS_UNROLL + 1):
                if writes[uu - 1] is not None:
                    for w in writes[uu - 1]:
                        w.wait()
            return _

        lax.fori_loop(0, n_chunks // _AS_UNROLL, block, 0)

    return k(tsub, tobj, sub_idx, obj_idx)


def kernel(wordembedding_corpus, rel_pair_idxs, prop_info, rel_info,
           W_sub, b_sub, W_obj, b_obj, W_int, b_int):
    idx = rel_pair_idxs.astype(jnp.int32)
    sub_idx = idx[:, 0]
    obj_idx = idx[:, 1]

    prop128 = jnp.pad(prop_info, ((0, 0), (0, _PROP_PAD - prop_info.shape[1])))
    ps, po = _sc_gather_props(prop128, sub_idx, obj_idx)

    tsub, tobj = _tc_tables(wordembedding_corpus, W_sub, b_sub, W_obj, b_obj)

    W1p = jnp.zeros((_PROP_PAD, HID), jnp.float32).at[:9].set(W_int[:9])
    W2p = jnp.zeros((_PROP_PAD, HID), jnp.float32).at[:9].set(W_int[9:18])
    W3p = jnp.zeros((8, HID), jnp.float32).at[:2].set(W_int[18:20])
    rel8 = jnp.pad(rel_info, ((0, 0), (0, 6)))

    partial_out = _sc_assemble(tsub, tobj, sub_idx, obj_idx)
    return _tc_pos_into(partial_out, ps, po, rel8, W1p, W2p, W3p, b_int)


# speedup vs baseline: 1.0166x; 1.0021x over previous
import functools

import jax
import jax.numpy as jnp
from jax import lax
from jax.experimental import pallas as pl
from jax.experimental.pallas import tpu as pltpu
from jax.experimental.pallas import tpu_sc as plsc

NUM_PROPS = 8192
NUM_RELS = 65536
EMB_DIM = 200
HID = 1024

NC = 2
NS = 16
NW = NC * NS

ROWS_PER_W = NUM_RELS // NW


_SG_CHUNK = 128
_PROP_PAD = 128


def _sc_gather_props(prop128, sub_idx, obj_idx):
    mesh = plsc.VectorSubcoreMesh(core_axis_name="c", subcore_axis_name="s")

    @functools.partial(
        pl.kernel,
        out_type=[
            jax.ShapeDtypeStruct((NUM_RELS, _PROP_PAD), jnp.float32),
            jax.ShapeDtypeStruct((NUM_RELS, _PROP_PAD), jnp.float32),
        ],
        mesh=mesh,
        scratch_types=[
            pltpu.VMEM((_SG_CHUNK,), jnp.int32),
            pltpu.VMEM((_SG_CHUNK,), jnp.int32),
            pltpu.VMEM((_SG_CHUNK, _PROP_PAD), jnp.float32),
            pltpu.VMEM((_SG_CHUNK, _PROP_PAD), jnp.float32),
            pltpu.SemaphoreType.DMA,
            pltpu.SemaphoreType.DMA,
        ],
    )
    def k(prop_hbm, sub_hbm, obj_hbm, osub_hbm, oobj_hbm,
          idxs_v, idxo_v, bufs_v, bufo_v, sem_s, sem_o):
        wid = lax.axis_index("s") * NC + lax.axis_index("c")
        base0 = wid * ROWS_PER_W

        def body(j, _):
            base = base0 + j * _SG_CHUNK
            pltpu.sync_copy(sub_hbm.at[pl.ds(base, _SG_CHUNK)], idxs_v)
            pltpu.sync_copy(obj_hbm.at[pl.ds(base, _SG_CHUNK)], idxo_v)
            cs = pltpu.async_copy(prop_hbm.at[idxs_v], bufs_v, sem_s)
            co = pltpu.async_copy(prop_hbm.at[idxo_v], bufo_v, sem_o)
            cs.wait()
            pltpu.sync_copy(bufs_v, osub_hbm.at[pl.ds(base, _SG_CHUNK)])
            co.wait()
            pltpu.sync_copy(bufo_v, oobj_hbm.at[pl.ds(base, _SG_CHUNK)])
            return _

        lax.fori_loop(0, ROWS_PER_W // _SG_CHUNK, body, 0)

    return k(prop128, sub_idx, obj_idx)


def _tc_tables(corpus, W_sub, b_sub, W_obj, b_obj):
    blk = 1024
    grid = NUM_PROPS // blk

    def body(x_ref, ws_ref, bs_ref, wo_ref, bo_ref, ts_ref, to_ref):
        x = x_ref[...]
        ts_ref[...] = jnp.maximum(
            jnp.dot(x, ws_ref[...], preferred_element_type=jnp.float32)
            + bs_ref[...],
            0.0,
        )
        to_ref[...] = jnp.maximum(
            jnp.dot(x, wo_ref[...], preferred_element_type=jnp.float32)
            + bo_ref[...],
            0.0,
        )

    return pl.pallas_call(
        body,
        grid=(grid,),
        in_specs=[
            pl.BlockSpec((blk, EMB_DIM), lambda i: (i, 0)),
            pl.BlockSpec((EMB_DIM, HID), lambda i: (0, 0)),
            pl.BlockSpec((1, HID), lambda i: (0, 0)),
            pl.BlockSpec((EMB_DIM, HID), lambda i: (0, 0)),
            pl.BlockSpec((1, HID), lambda i: (0, 0)),
        ],
        out_specs=[
            pl.BlockSpec((blk, HID), lambda i: (i, 0)),
            pl.BlockSpec((blk, HID), lambda i: (i, 0)),
        ],
        out_shape=[
            jax.ShapeDtypeStruct((NUM_PROPS, HID), jnp.float32),
            jax.ShapeDtypeStruct((NUM_PROPS, HID), jnp.float32),
        ],
    )(corpus, W_sub, b_sub.reshape(1, HID), W_obj, b_obj.reshape(1, HID))


def _tc_pos_into(partial_out, ps, po, rel8, W1p, W2p, W3p, b_int):
    blk = 2048
    grid = NUM_RELS // blk

    def body(buf_ref, x1_ref, x2_ref, x3_ref, w1_ref, w2_ref, w3_ref, b_ref,
             o_ref, acc_ref, sem):
        i = pl.program_id(0)
        slot = lax.rem(i, 2)

        @pl.when(i >= 2)
        def _():
            pltpu.make_async_copy(
                acc_ref.at[slot],
                o_ref.at[pl.ds((i - 2) * blk, blk), pl.ds(2 * HID, HID)],
                sem,
            ).wait()

        acc = jnp.dot(x1_ref[...], w1_ref[...], preferred_element_type=jnp.float32)
        acc += jnp.dot(x2_ref[...], w2_ref[...], preferred_element_type=jnp.float32)
        acc += jnp.dot(x3_ref[...], w3_ref[...], preferred_element_type=jnp.float32)
        acc_ref[slot] = jnp.maximum(acc + b_ref[...], 0.0)

        pltpu.make_async_copy(
            acc_ref.at[slot],
            o_ref.at[pl.ds(i * blk, blk), pl.ds(2 * HID, HID)],
            sem,
        ).start()

        @pl.when(i == grid - 1)
        def _():
            for back in (1, 0):
                pltpu.make_async_copy(
                    acc_ref.at[slot],
                    o_ref.at[pl.ds((i - back) * blk, blk), pl.ds(2 * HID, HID)],
                    sem,
                ).wait()

    return pl.pallas_call(
        body,
        grid=(grid,),
        in_specs=[
            pl.BlockSpec(memory_space=pl.ANY),
            pl.BlockSpec((blk, _PROP_PAD), lambda i: (i, 0)),
            pl.BlockSpec((blk, _PROP_PAD), lambda i: (i, 0)),
            pl.BlockSpec((blk, 8), lambda i: (i, 0)),
            pl.BlockSpec((_PROP_PAD, HID), lambda i: (0, 0)),
            pl.BlockSpec((_PROP_PAD, HID), lambda i: (0, 0)),
            pl.BlockSpec((8, HID), lambda i: (0, 0)),
            pl.BlockSpec((1, HID), lambda i: (0, 0)),
        ],
        out_specs=pl.BlockSpec(memory_space=pl.ANY),
        out_shape=jax.ShapeDtypeStruct((NUM_RELS, 3 * HID), jnp.float32),
        scratch_shapes=[
            pltpu.VMEM((2, blk, HID), jnp.float32),
            pltpu.SemaphoreType.DMA,
        ],
        input_output_aliases={0: 0},
    )(partial_out, ps, po, rel8, W1p, W2p, W3p, b_int.reshape(1, HID))


_AS_CHUNK = 16
_AS_UNROLL = 16


def _sc_assemble(tsub, tobj, sub_idx, obj_idx):
    mesh = plsc.VectorSubcoreMesh(core_axis_name="c", subcore_axis_name="s")
    n_chunks = ROWS_PER_W // _AS_CHUNK
    NSLOT = 3

    @functools.partial(
        pl.kernel,
        out_type=jax.ShapeDtypeStruct((NUM_RELS, 3 * HID), jnp.float32),
        mesh=mesh,
        scratch_types=[
            pltpu.VMEM((ROWS_PER_W,), jnp.int32),
            pltpu.VMEM((ROWS_PER_W,), jnp.int32),
            pltpu.VMEM((NSLOT, _AS_CHUNK, 2 * HID), jnp.float32),
        ] + [pltpu.SemaphoreType.DMA] * (3 * NSLOT),
    )
    def k(tsub_hbm, tobj_hbm, sub_hbm, obj_hbm, out_hbm,
          idxs_v, idxo_v, b_v, *sems):
        wid = lax.axis_index("s") * NC + lax.axis_index("c")
        base0 = wid * ROWS_PER_W
        pltpu.sync_copy(sub_hbm.at[pl.ds(base0, ROWS_PER_W)], idxs_v)
        pltpu.sync_copy(obj_hbm.at[pl.ds(base0, ROWS_PER_W)], idxo_v)
        gsem = sems[0:NSLOT]
        osem = sems[NSLOT:2 * NSLOT]
        wsem = sems[2 * NSLOT:3 * NSLOT]

        def fire_gather(j, s):
            off = j * _AS_CHUNK
            return (
                pltpu.async_copy(
                    tsub_hbm.at[idxs_v.at[pl.ds(off, _AS_CHUNK)]],
                    b_v.at[s, :, pl.ds(0, HID)], gsem[s]),
                pltpu.async_copy(
                    tobj_hbm.at[idxo_v.at[pl.ds(off, _AS_CHUNK)]],
                    b_v.at[s, :, pl.ds(HID, HID)], osem[s]),
            )

        def fire_write(j, s):
            base = base0 + j * _AS_CHUNK
            return (
                pltpu.async_copy(
                    b_v.at[s],
                    out_hbm.at[pl.ds(base, _AS_CHUNK), pl.ds(0, 2 * HID)],
                    wsem[s]),
            )

        def block(g, _):
            j0 = g * _AS_UNROLL
            gathers = [None] * _AS_UNROLL
            writes = [None] * _AS_UNROLL
            for u in range(_AS_UNROLL):
                s = u % NSLOT
                if u >= NSLOT:
                    for w in writes[u - NSLOT]:
                        w.wait()
                gathers[u] = fire_gather(j0 + u, s)
                if u >= 1:
                    for gcp in gathers[u - 1]:
                        gcp.wait()
                    writes[u - 1] = fire_write(j0 + u - 1, (u - 1) % NSLOT)
            u = _AS_UNROLL - 1
            for gcp in gathers[u]:
                gcp.wait()
            writes[u] = fire_write(j0 + u, u % NSLOT)
            for uu in range(_AS_UNROLL - NSLOT + 1, _AS_UNROLL + 1):
                if writes[uu - 1] is not None:
                    for w in writes[uu - 1]:
                        w.wait()
            return _

        lax.fori_loop(0, n_chunks // _AS_UNROLL, block, 0)

    return k(tsub, tobj, sub_idx, obj_idx)


def kernel(wordembedding_corpus, rel_pair_idxs, prop_info, rel_info,
           W_sub, b_sub, W_obj, b_obj, W_int, b_int):
    idx = rel_pair_idxs.astype(jnp.int32)
    sub_idx = idx[:, 0]
    obj_idx = idx[:, 1]

    prop128 = jnp.pad(prop_info, ((0, 0), (0, _PROP_PAD - prop_info.shape[1])))
    ps, po = _sc_gather_props(prop128, sub_idx, obj_idx)

    tsub, tobj = _tc_tables(wordembedding_corpus, W_sub, b_sub, W_obj, b_obj)

    W1p = jnp.zeros((_PROP_PAD, HID), jnp.float32).at[:9].set(W_int[:9])
    W2p = jnp.zeros((_PROP_PAD, HID), jnp.float32).at[:9].set(W_int[9:18])
    W3p = jnp.zeros((8, HID), jnp.float32).at[:2].set(W_int[18:20])
    rel8 = jnp.pad(rel_info, ((0, 0), (0, 6)))

    partial_out = _sc_assemble(tsub, tobj, sub_idx, obj_idx)
    return _tc_pos_into(partial_out, ps, po, rel8, W1p, W2p, W3p, b_int)


# speedup vs baseline: 1.0183x; 1.0017x over previous
import functools

import jax
import jax.numpy as jnp
from jax import lax
from jax.experimental import pallas as pl
from jax.experimental.pallas import tpu as pltpu
from jax.experimental.pallas import tpu_sc as plsc

NUM_PROPS = 8192
NUM_RELS = 65536
EMB_DIM = 200
HID = 1024

NC = 2
NS = 16
NW = NC * NS

ROWS_PER_W = NUM_RELS // NW


_SG_CHUNK = 128
_PROP_PAD = 128


def _sc_gather_props(prop128, sub_idx, obj_idx):
    mesh = plsc.VectorSubcoreMesh(core_axis_name="c", subcore_axis_name="s")
    n_chunks = ROWS_PER_W // _SG_CHUNK

    @functools.partial(
        pl.kernel,
        out_type=[
            jax.ShapeDtypeStruct((NUM_RELS, _PROP_PAD), jnp.float32),
            jax.ShapeDtypeStruct((NUM_RELS, _PROP_PAD), jnp.float32),
        ],
        mesh=mesh,
        scratch_types=[
            pltpu.VMEM((ROWS_PER_W,), jnp.int32),
            pltpu.VMEM((ROWS_PER_W,), jnp.int32),
            pltpu.VMEM((2, _SG_CHUNK, _PROP_PAD), jnp.float32),
            pltpu.VMEM((2, _SG_CHUNK, _PROP_PAD), jnp.float32),
        ] + [pltpu.SemaphoreType.DMA] * 8,
    )
    def k(prop_hbm, sub_hbm, obj_hbm, osub_hbm, oobj_hbm,
          idxs_v, idxo_v, bufs_v, bufo_v, *sems):
        wid = lax.axis_index("s") * NC + lax.axis_index("c")
        base0 = wid * ROWS_PER_W
        pltpu.sync_copy(sub_hbm.at[pl.ds(base0, ROWS_PER_W)], idxs_v)
        pltpu.sync_copy(obj_hbm.at[pl.ds(base0, ROWS_PER_W)], idxo_v)
        gs, go = sems[0:2], sems[2:4]
        ws, wo = sems[4:6], sems[6:8]

        def fire_gather(j, slot):
            off = j * _SG_CHUNK
            return (
                pltpu.async_copy(prop_hbm.at[idxs_v.at[pl.ds(off, _SG_CHUNK)]],
                                 bufs_v.at[slot], gs[slot]),
                pltpu.async_copy(prop_hbm.at[idxo_v.at[pl.ds(off, _SG_CHUNK)]],
                                 bufo_v.at[slot], go[slot]),
            )

        def fire_write(j, slot):
            base = base0 + j * _SG_CHUNK
            return (
                pltpu.async_copy(bufs_v.at[slot],
                                 osub_hbm.at[pl.ds(base, _SG_CHUNK)], ws[slot]),
                pltpu.async_copy(bufo_v.at[slot],
                                 oobj_hbm.at[pl.ds(base, _SG_CHUNK)], wo[slot]),
            )

        gathers = [None] * n_chunks
        writes = [None] * n_chunks
        for u in range(n_chunks):
            slot = u % 2
            if u >= 2:
                for w in writes[u - 2]:
                    w.wait()
            gathers[u] = fire_gather(u, slot)
            if u >= 1:
                for g in gathers[u - 1]:
                    g.wait()
                writes[u - 1] = fire_write(u - 1, (u - 1) % 2)
        u = n_chunks - 1
        for g in gathers[u]:
            g.wait()
        writes[u] = fire_write(u, u % 2)
        for w in writes[u - 1]:
            w.wait()
        for w in writes[u]:
            w.wait()

    return k(prop128, sub_idx, obj_idx)


def _tc_tables(corpus, W_sub, b_sub, W_obj, b_obj):
    blk = 1024
    grid = NUM_PROPS // blk

    def body(x_ref, ws_ref, bs_ref, wo_ref, bo_ref, ts_ref, to_ref):
        x = x_ref[...]
        ts_ref[...] = jnp.maximum(
            jnp.dot(x, ws_ref[...], preferred_element_type=jnp.float32)
            + bs_ref[...],
            0.0,
        )
        to_ref[...] = jnp.maximum(
            jnp.dot(x, wo_ref[...], preferred_element_type=jnp.float32)
            + bo_ref[...],
            0.0,
        )

    return pl.pallas_call(
        body,
        grid=(grid,),
        in_specs=[
            pl.BlockSpec((blk, EMB_DIM), lambda i: (i, 0)),
            pl.BlockSpec((EMB_DIM, HID), lambda i: (0, 0)),
            pl.BlockSpec((1, HID), lambda i: (0, 0)),
            pl.BlockSpec((EMB_DIM, HID), lambda i: (0, 0)),
            pl.BlockSpec((1, HID), lambda i: (0, 0)),
        ],
        out_specs=[
            pl.BlockSpec((blk, HID), lambda i: (i, 0)),
            pl.BlockSpec((blk, HID), lambda i: (i, 0)),
        ],
        out_shape=[
            jax.ShapeDtypeStruct((NUM_PROPS, HID), jnp.float32),
            jax.ShapeDtypeStruct((NUM_PROPS, HID), jnp.float32),
        ],
    )(corpus, W_sub, b_sub.reshape(1, HID), W_obj, b_obj.reshape(1, HID))


def _tc_pos_into(partial_out, ps, po, rel8, W1p, W2p, W3p, b_int):
    blk = 2048
    grid = NUM_RELS // blk

    def body(buf_ref, x1_ref, x2_ref, x3_ref, w1_ref, w2_ref, w3_ref, b_ref,
             o_ref, acc_ref, sem):
        i = pl.program_id(0)
        slot = lax.rem(i, 2)

        @pl.when(i >= 2)
        def _():
            pltpu.make_async_copy(
                acc_ref.at[slot],
                o_ref.at[pl.ds((i - 2) * blk, blk), pl.ds(2 * HID, HID)],
                sem,
            ).wait()

        acc = jnp.dot(x1_ref[...], w1_ref[...], preferred_element_type=jnp.float32)
        acc += jnp.dot(x2_ref[...], w2_ref[...], preferred_element_type=jnp.float32)
        acc += jnp.dot(x3_ref[...], w3_ref[...], preferred_element_type=jnp.float32)
        acc_ref[slot] = jnp.maximum(acc + b_ref[...], 0.0)

        pltpu.make_async_copy(
            acc_ref.at[slot],
            o_ref.at[pl.ds(i * blk, blk), pl.ds(2 * HID, HID)],
            sem,
        ).start()

        @pl.when(i == grid - 1)
        def _():
            for back in (1, 0):
                pltpu.make_async_copy(
                    acc_ref.at[slot],
                    o_ref.at[pl.ds((i - back) * blk, blk), pl.ds(2 * HID, HID)],
                    sem,
                ).wait()

    return pl.pallas_call(
        body,
        grid=(grid,),
        in_specs=[
            pl.BlockSpec(memory_space=pl.ANY),
            pl.BlockSpec((blk, _PROP_PAD), lambda i: (i, 0)),
            pl.BlockSpec((blk, _PROP_PAD), lambda i: (i, 0)),
            pl.BlockSpec((blk, 8), lambda i: (i, 0)),
            pl.BlockSpec((_PROP_PAD, HID), lambda i: (0, 0)),
            pl.BlockSpec((_PROP_PAD, HID), lambda i: (0, 0)),
            pl.BlockSpec((8, HID), lambda i: (0, 0)),
            pl.BlockSpec((1, HID), lambda i: (0, 0)),
        ],
        out_specs=pl.BlockSpec(memory_space=pl.ANY),
        out_shape=jax.ShapeDtypeStruct((NUM_RELS, 3 * HID), jnp.float32),
        scratch_shapes=[
            pltpu.VMEM((2, blk, HID), jnp.float32),
            pltpu.SemaphoreType.DMA,
        ],
        input_output_aliases={0: 0},
    )(partial_out, ps, po, rel8, W1p, W2p, W3p, b_int.reshape(1, HID))


_AS_CHUNK = 16
_AS_UNROLL = 16


def _sc_assemble(tsub, tobj, sub_idx, obj_idx):
    mesh = plsc.VectorSubcoreMesh(core_axis_name="c", subcore_axis_name="s")
    n_chunks = ROWS_PER_W // _AS_CHUNK
    NSLOT = 3

    @functools.partial(
        pl.kernel,
        out_type=jax.ShapeDtypeStruct((NUM_RELS, 3 * HID), jnp.float32),
        mesh=mesh,
        scratch_types=[
            pltpu.VMEM((ROWS_PER_W,), jnp.int32),
            pltpu.VMEM((ROWS_PER_W,), jnp.int32),
            pltpu.VMEM((NSLOT, _AS_CHUNK, 2 * HID), jnp.float32),
        ] + [pltpu.SemaphoreType.DMA] * (3 * NSLOT),
    )
    def k(tsub_hbm, tobj_hbm, sub_hbm, obj_hbm, out_hbm,
          idxs_v, idxo_v, b_v, *sems):
        wid = lax.axis_index("s") * NC + lax.axis_index("c")
        base0 = wid * ROWS_PER_W
        pltpu.sync_copy(sub_hbm.at[pl.ds(base0, ROWS_PER_W)], idxs_v)
        pltpu.sync_copy(obj_hbm.at[pl.ds(base0, ROWS_PER_W)], idxo_v)
        gsem = sems[0:NSLOT]
        osem = sems[NSLOT:2 * NSLOT]
        wsem = sems[2 * NSLOT:3 * NSLOT]

        def fire_gather(j, s):
            off = j * _AS_CHUNK
            return (
                pltpu.async_copy(
                    tsub_hbm.at[idxs_v.at[pl.ds(off, _AS_CHUNK)]],
                    b_v.at[s, :, pl.ds(0, HID)], gsem[s]),
                pltpu.async_copy(
                    tobj_hbm.at[idxo_v.at[pl.ds(off, _AS_CHUNK)]],
                    b_v.at[s, :, pl.ds(HID, HID)], osem[s]),
            )

        def fire_write(j, s):
            base = base0 + j * _AS_CHUNK
            return (
                pltpu.async_copy(
                    b_v.at[s],
                    out_hbm.at[pl.ds(base, _AS_CHUNK), pl.ds(0, 2 * HID)],
                    wsem[s]),
            )

        def block(g, _):
            j0 = g * _AS_UNROLL
            gathers = [None] * _AS_UNROLL
            writes = [None] * _AS_UNROLL
            for u in range(_AS_UNROLL):
                s = u % NSLOT
                if u >= NSLOT:
                    for w in writes[u - NSLOT]:
                        w.wait()
                gathers[u] = fire_gather(j0 + u, s)
                if u >= 1:
                    for gcp in gathers[u - 1]:
                        gcp.wait()
                    writes[u - 1] = fire_write(j0 + u - 1, (u - 1) % NSLOT)
            u = _AS_UNROLL - 1
            for gcp in gathers[u]:
                gcp.wait()
            writes[u] = fire_write(j0 + u, u % NSLOT)
            for uu in range(_AS_UNROLL - NSLOT + 1, _AS_UNROLL + 1):
                if writes[uu - 1] is not None:
                    for w in writes[uu - 1]:
                        w.wait()
            return _

        lax.fori_loop(0, n_chunks // _AS_UNROLL, block, 0)

    return k(tsub, tobj, sub_idx, obj_idx)


def kernel(wordembedding_corpus, rel_pair_idxs, prop_info, rel_info,
           W_sub, b_sub, W_obj, b_obj, W_int, b_int):
    idx = rel_pair_idxs.astype(jnp.int32)
    sub_idx = idx[:, 0]
    obj_idx = idx[:, 1]

    prop128 = jnp.pad(prop_info, ((0, 0), (0, _PROP_PAD - prop_info.shape[1])))
    ps, po = _sc_gather_props(prop128, sub_idx, obj_idx)

    tsub, tobj = _tc_tables(wordembedding_corpus, W_sub, b_sub, W_obj, b_obj)

    W1p = jnp.zeros((_PROP_PAD, HID), jnp.float32).at[:9].set(W_int[:9])
    W2p = jnp.zeros((_PROP_PAD, HID), jnp.float32).at[:9].set(W_int[9:18])
    W3p = jnp.zeros((8, HID), jnp.float32).at[:2].set(W_int[18:20])
    rel8 = jnp.pad(rel_info, ((0, 0), (0, 6)))

    partial_out = _sc_assemble(tsub, tobj, sub_idx, obj_idx)
    return _tc_pos_into(partial_out, ps, po, rel8, W1p, W2p, W3p, b_int)


# speedup vs baseline: 1.0552x; 1.0362x over previous
import functools

import jax
import jax.numpy as jnp
from jax import lax
from jax.experimental import pallas as pl
from jax.experimental.pallas import tpu as pltpu
from jax.experimental.pallas import tpu_sc as plsc

NUM_PROPS = 8192
NUM_RELS = 65536
EMB_DIM = 200
HID = 1024

NC = 2
NS = 16
NW = NC * NS

ROWS_PER_W = NUM_RELS // NW


_SG_CHUNK = 128
_PROP_PAD = 128


def _sc_gather_props(prop128, sub_idx, obj_idx):
    mesh = plsc.VectorSubcoreMesh(core_axis_name="c", subcore_axis_name="s")
    n_chunks = ROWS_PER_W // _SG_CHUNK

    @functools.partial(
        pl.kernel,
        out_type=[
            jax.ShapeDtypeStruct((NUM_RELS, _PROP_PAD), jnp.float32),
            jax.ShapeDtypeStruct((NUM_RELS, _PROP_PAD), jnp.float32),
        ],
        mesh=mesh,
        scratch_types=[
            pltpu.VMEM((ROWS_PER_W,), jnp.int32),
            pltpu.VMEM((ROWS_PER_W,), jnp.int32),
            pltpu.VMEM((2, _SG_CHUNK, _PROP_PAD), jnp.float32),
            pltpu.VMEM((2, _SG_CHUNK, _PROP_PAD), jnp.float32),
        ] + [pltpu.SemaphoreType.DMA] * 8,
    )
    def k(prop_hbm, sub_hbm, obj_hbm, osub_hbm, oobj_hbm,
          idxs_v, idxo_v, bufs_v, bufo_v, *sems):
        wid = lax.axis_index("s") * NC + lax.axis_index("c")
        base0 = wid * ROWS_PER_W
        pltpu.sync_copy(sub_hbm.at[pl.ds(base0, ROWS_PER_W)], idxs_v)
        pltpu.sync_copy(obj_hbm.at[pl.ds(base0, ROWS_PER_W)], idxo_v)
        gs, go = sems[0:2], sems[2:4]
        ws, wo = sems[4:6], sems[6:8]

        def fire_gather(j, slot):
            off = j * _SG_CHUNK
            return (
                pltpu.async_copy(prop_hbm.at[idxs_v.at[pl.ds(off, _SG_CHUNK)]],
                                 bufs_v.at[slot], gs[slot]),
                pltpu.async_copy(prop_hbm.at[idxo_v.at[pl.ds(off, _SG_CHUNK)]],
                                 bufo_v.at[slot], go[slot]),
            )

        def fire_write(j, slot):
            base = base0 + j * _SG_CHUNK
            return (
                pltpu.async_copy(bufs_v.at[slot],
                                 osub_hbm.at[pl.ds(base, _SG_CHUNK)], ws[slot]),
                pltpu.async_copy(bufo_v.at[slot],
                                 oobj_hbm.at[pl.ds(base, _SG_CHUNK)], wo[slot]),
            )

        gathers = [None] * n_chunks
        writes = [None] * n_chunks
        for u in range(n_chunks):
            slot = u % 2
            if u >= 2:
                for w in writes[u - 2]:
                    w.wait()
            gathers[u] = fire_gather(u, slot)
            if u >= 1:
                for g in gathers[u - 1]:
                    g.wait()
                writes[u - 1] = fire_write(u - 1, (u - 1) % 2)
        u = n_chunks - 1
        for g in gathers[u]:
            g.wait()
        writes[u] = fire_write(u, u % 2)
        for w in writes[u - 1]:
            w.wait()
        for w in writes[u]:
            w.wait()

    return k(prop128, sub_idx, obj_idx)


def _tc_tables(corpus, W_sub, b_sub, W_obj, b_obj):
    blk = 1024
    grid = NUM_PROPS // blk

    def body(x_ref, ws_ref, bs_ref, wo_ref, bo_ref, ts_ref, to_ref):
        x = x_ref[...]
        ts_ref[...] = jnp.maximum(
            jnp.dot(x, ws_ref[...], preferred_element_type=jnp.float32)
            + bs_ref[...],
            0.0,
        )
        to_ref[...] = jnp.maximum(
            jnp.dot(x, wo_ref[...], preferred_element_type=jnp.float32)
            + bo_ref[...],
            0.0,
        )

    return pl.pallas_call(
        body,
        grid=(grid,),
        in_specs=[
            pl.BlockSpec((blk, EMB_DIM), lambda i: (i, 0)),
            pl.BlockSpec((EMB_DIM, HID), lambda i: (0, 0)),
            pl.BlockSpec((1, HID), lambda i: (0, 0)),
            pl.BlockSpec((EMB_DIM, HID), lambda i: (0, 0)),
            pl.BlockSpec((1, HID), lambda i: (0, 0)),
        ],
        out_specs=[
            pl.BlockSpec((blk, HID), lambda i: (i, 0)),
            pl.BlockSpec((blk, HID), lambda i: (i, 0)),
        ],
        out_shape=[
            jax.ShapeDtypeStruct((NUM_PROPS, HID), jnp.float32),
            jax.ShapeDtypeStruct((NUM_PROPS, HID), jnp.float32),
        ],
    )(corpus, W_sub, b_sub.reshape(1, HID), W_obj, b_obj.reshape(1, HID))


def _tc_pos_into(partial_out, ps, po, rel8, W1p, W2p, W3p, b_int):
    blk = 2048
    grid = NUM_RELS // blk

    def body(buf_ref, x1_ref, x2_ref, x3_ref, w1_ref, w2_ref, w3_ref, b_ref,
             o_ref, acc_ref, sem):
        i = pl.program_id(0)
        slot = lax.rem(i, 2)

        @pl.when(i >= 2)
        def _():
            pltpu.make_async_copy(
                acc_ref.at[slot],
                o_ref.at[pl.ds((i - 2) * blk, blk), pl.ds(2 * HID, HID)],
                sem,
            ).wait()

        acc = jnp.dot(x1_ref[...], w1_ref[...], preferred_element_type=jnp.float32)
        acc += jnp.dot(x2_ref[...], w2_ref[...], preferred_element_type=jnp.float32)
        acc += jnp.dot(x3_ref[...], w3_ref[...], preferred_element_type=jnp.float32)
        acc_ref[slot] = jnp.maximum(acc + b_ref[...], 0.0)

        pltpu.make_async_copy(
            acc_ref.at[slot],
            o_ref.at[pl.ds(i * blk, blk), pl.ds(2 * HID, HID)],
            sem,
        ).start()

        @pl.when(i == grid - 1)
        def _():
            for back in (1, 0):
                pltpu.make_async_copy(
                    acc_ref.at[slot],
                    o_ref.at[pl.ds((i - back) * blk, blk), pl.ds(2 * HID, HID)],
                    sem,
                ).wait()

    return pl.pallas_call(
        body,
        grid=(grid,),
        in_specs=[
            pl.BlockSpec(memory_space=pl.ANY),
            pl.BlockSpec((blk, _PROP_PAD), lambda i: (i, 0)),
            pl.BlockSpec((blk, _PROP_PAD), lambda i: (i, 0)),
            pl.BlockSpec((blk, 2), lambda i: (i, 0)),
            pl.BlockSpec((_PROP_PAD, HID), lambda i: (0, 0)),
            pl.BlockSpec((_PROP_PAD, HID), lambda i: (0, 0)),
            pl.BlockSpec((2, HID), lambda i: (0, 0)),
            pl.BlockSpec((1, HID), lambda i: (0, 0)),
        ],
        out_specs=pl.BlockSpec(memory_space=pl.ANY),
        out_shape=jax.ShapeDtypeStruct((NUM_RELS, 3 * HID), jnp.float32),
        scratch_shapes=[
            pltpu.VMEM((2, blk, HID), jnp.float32),
            pltpu.SemaphoreType.DMA,
        ],
        input_output_aliases={0: 0},
    )(partial_out, ps, po, rel8, W1p, W2p, W3p, b_int.reshape(1, HID))


_AS_CHUNK = 16
_AS_UNROLL = 16


def _sc_assemble(tsub, tobj, sub_idx, obj_idx):
    mesh = plsc.VectorSubcoreMesh(core_axis_name="c", subcore_axis_name="s")
    n_chunks = ROWS_PER_W // _AS_CHUNK
    NSLOT = 3

    @functools.partial(
        pl.kernel,
        out_type=jax.ShapeDtypeStruct((NUM_RELS, 3 * HID), jnp.float32),
        mesh=mesh,
        scratch_types=[
            pltpu.VMEM((ROWS_PER_W,), jnp.int32),
            pltpu.VMEM((ROWS_PER_W,), jnp.int32),
            pltpu.VMEM((NSLOT, _AS_CHUNK, 2 * HID), jnp.float32),
        ] + [pltpu.SemaphoreType.DMA] * (3 * NSLOT),
    )
    def k(tsub_hbm, tobj_hbm, sub_hbm, obj_hbm, out_hbm,
          idxs_v, idxo_v, b_v, *sems):
        wid = lax.axis_index("s") * NC + lax.axis_index("c")
        base0 = wid * ROWS_PER_W
        pltpu.sync_copy(sub_hbm.at[pl.ds(base0, ROWS_PER_W)], idxs_v)
        pltpu.sync_copy(obj_hbm.at[pl.ds(base0, ROWS_PER_W)], idxo_v)
        gsem = sems[0:NSLOT]
        osem = sems[NSLOT:2 * NSLOT]
        wsem = sems[2 * NSLOT:3 * NSLOT]

        def fire_gather(j, s):
            off = j * _AS_CHUNK
            return (
                pltpu.async_copy(
                    tsub_hbm.at[idxs_v.at[pl.ds(off, _AS_CHUNK)]],
                    b_v.at[s, :, pl.ds(0, HID)], gsem[s]),
                pltpu.async_copy(
                    tobj_hbm.at[idxo_v.at[pl.ds(off, _AS_CHUNK)]],
                    b_v.at[s, :, pl.ds(HID, HID)], osem[s]),
            )

        def fire_write(j, s):
            base = base0 + j * _AS_CHUNK
            return (
                pltpu.async_copy(
                    b_v.at[s],
                    out_hbm.at[pl.ds(base, _AS_CHUNK), pl.ds(0, 2 * HID)],
                    wsem[s]),
            )

        def block(g, _):
            j0 = g * _AS_UNROLL
            gathers = [None] * _AS_UNROLL
            writes = [None] * _AS_UNROLL
            for u in range(_AS_UNROLL):
                s = u % NSLOT
                if u >= NSLOT:
                    for w in writes[u - NSLOT]:
                        w.wait()
                gathers[u] = fire_gather(j0 + u, s)
                if u >= 1:
                    for gcp in gathers[u - 1]:
                        gcp.wait()
                    writes[u - 1] = fire_write(j0 + u - 1, (u - 1) % NSLOT)
            u = _AS_UNROLL - 1
            for gcp in gathers[u]:
                gcp.wait()
            writes[u] = fire_write(j0 + u, u % NSLOT)
            for uu in range(_AS_UNROLL - NSLOT + 1, _AS_UNROLL + 1):
                if writes[uu - 1] is not None:
                    for w in writes[uu - 1]:
                        w.wait()
            return _

        lax.fori_loop(0, n_chunks // _AS_UNROLL, block, 0)

    return k(tsub, tobj, sub_idx, obj_idx)


def kernel(wordembedding_corpus, rel_pair_idxs, prop_info, rel_info,
           W_sub, b_sub, W_obj, b_obj, W_int, b_int):
    idx = rel_pair_idxs.astype(jnp.int32)
    sub_idx = idx[:, 0]
    obj_idx = idx[:, 1]

    prop128 = jnp.pad(prop_info, ((0, 0), (0, _PROP_PAD - prop_info.shape[1])))
    ps, po = _sc_gather_props(prop128, sub_idx, obj_idx)

    tsub, tobj = _tc_tables(wordembedding_corpus, W_sub, b_sub, W_obj, b_obj)

    W1p = jnp.zeros((_PROP_PAD, HID), jnp.float32).at[:9].set(W_int[:9])
    W2p = jnp.zeros((_PROP_PAD, HID), jnp.float32).at[:9].set(W_int[9:18])
    W3 = W_int[18:20]

    partial_out = _sc_assemble(tsub, tobj, sub_idx, obj_idx)
    return _tc_pos_into(partial_out, ps, po, rel_info, W1p, W2p, W3, b_int)
